# baseline (speedup 1.0000x reference)
import functools

import jax
import jax.numpy as jnp
from jax import lax
from jax.experimental import pallas as pl
from jax.experimental.pallas import tpu as pltpu
from jax.experimental.pallas import tpu_sc as plsc


_K = 16
_NC = 2
_NS = 16
_NW = _NC * _NS


def _sc_stage1(ux, uy, x, y, z, B, M):
    N = B * M
    ppt = N // _NW
    seg = M // ppt
    nchunk = M // 16

    @functools.partial(
        pl.kernel,
        mesh=plsc.VectorSubcoreMesh(core_axis_name="c", subcore_axis_name="s"),
        out_type=(jax.ShapeDtypeStruct((16 * N,), jnp.float32),
                  jax.ShapeDtypeStruct((16 * N,), jnp.float32),
                  jax.ShapeDtypeStruct((16 * N,), jnp.float32)),
        compiler_params=pltpu.CompilerParams(needs_layout_passes=False),
        scratch_types=[
            pltpu.VMEM((M,), jnp.float32),
            pltpu.VMEM((M,), jnp.float32),
            pltpu.VMEM((M,), jnp.float32),
            pltpu.VMEM((M,), jnp.float32),
            pltpu.VMEM((M,), jnp.float32),
            pltpu.VMEM((16 * ppt,), jnp.float32),
            pltpu.VMEM((16 * ppt,), jnp.float32),
            pltpu.VMEM((16 * ppt,), jnp.float32),
            pltpu.VMEM((M + 128,), jnp.int32),
            pltpu.SMEM((16,), jnp.int32),
            pltpu.SMEM((16,), jnp.int32),
        ],
    )
    def body(ux_hbm, uy_hbm, x_hbm, y_hbm, z_hbm,
             ox_hbm, oy_hbm, oz_hbm,
             ux_v, uy_v, x_v, y_v, z_v, nx_v, ny_v, nz_v,
             bin_v, st_s, cn_s):
        cid = lax.axis_index("c")
        sid = lax.axis_index("s")
        wid = sid * _NC + cid
        base = wid * ppt
        boff = (wid // seg) * M
        pltpu.sync_copy(ux_hbm.at[pl.ds(boff, M)], ux_v)
        pltpu.sync_copy(uy_hbm.at[pl.ds(boff, M)], uy_v)
        pltpu.sync_copy(x_hbm.at[pl.ds(boff, M)], x_v)
        pltpu.sync_copy(y_hbm.at[pl.ds(boff, M)], y_v)
        pltpu.sync_copy(z_hbm.at[pl.ds(boff, M)], z_v)

        lane = lax.iota(jnp.int32, 16)
        lane_ppt = lane * ppt
        zero16 = jnp.zeros((16,), jnp.int32)
        inf16 = jnp.full((16,), jnp.float32(jnp.inf))

        def merge(a, b, descending):
            take = b[0] < a[0]
            nd = jnp.where(take, b[0], a[0])
            ni = jnp.where(take, b[1], a[1])
            nk, nv = plsc.sort_key_val(nd, ni, descending=descending)
            return (nk, nv)

        finf = jnp.float32(jnp.inf)
        scale = jnp.float32(16.0)
        inv16 = jnp.float32(1.0 / 16.0)

        for t in range(8):
            bin_v[pl.ds(M + t * 16, 16)] = zero16

        off0 = jnp.int32(0)
        for s in range(16):
            st_s[s] = off0

            def bin_chunk(c, off, s=s):
                u = ux_v[pl.ds(c * 16, 16)]
                sid = jnp.clip((u * scale).astype(jnp.int32), 0, 15)
                m = sid == s
                plsc.store_compressed(bin_v.at[pl.ds(off, 16)],
                                      lane + c * 16, mask=m)
                return off + plsc.all_reduce_population_count(m)[0]

            off1 = lax.fori_loop(0, nchunk, bin_chunk, off0)
            cn_s[s] = off1 - off0
            off0 = off1

        def strip_pass(t, carry, qx, qy):
            st = st_s[t]
            cn = cn_s[t]
            cnv = jnp.full((16,), cn, jnp.int32)

            def group(g, carry2):
                rel0 = g * 128
                level = []
                for j in range(8):
                    rel = rel0 + j * 16
                    idx = bin_v[pl.ds(st + rel, 16)]
                    valid = (lane + rel) < cnv
                    cu = plsc.load_gather(ux_v, [idx])
                    cv = plsc.load_gather(uy_v, [idx])
                    du = cu - qx
                    dv = cv - qy
                    d = du * du + dv * dv
                    d = jnp.where(valid, d, finf)
                    sd, si = plsc.sort_key_val(d, idx,
                                               descending=(j % 2 == 1))
                    level.append((sd, si))
                while len(level) > 2:
                    level = [merge(level[i], level[i + 1],
                                   descending=(i % 4 == 2))
                             for i in range(0, len(level), 2)]
                root = merge(level[0], level[1], descending=True)
                return merge(carry2, root, descending=False)

            return lax.fori_loop(0, (cn + 127) // 128, group, carry)

        def point_body(p, _):
            loc16 = jnp.full((16,), (wid % seg) * ppt + p, jnp.int32)
            qx = plsc.load_gather(ux_v, [loc16])
            qy = plsc.load_gather(uy_v, [loc16])
            qu = qx[0]
            s0 = jnp.clip((qu * scale).astype(jnp.int32), 0, 15)

            bd, bi = strip_pass(s0, (inf16, zero16), qx, qy)

            def bounds(jlo, jhi):
                bl = qu - jlo.astype(jnp.float32) * inv16
                br = (jhi + 1).astype(jnp.float32) * inv16 - qu
                bl2 = jnp.where(jlo > 0, bl * bl, finf)
                br2 = jnp.where(jhi < 15, br * br, finf)
                return bl2, br2

            def cond(carry):
                (bd, _), jlo, jhi = carry
                bl2, br2 = bounds(jlo, jhi)
                return bd[15] > jnp.minimum(bl2, br2)

            def expand(carry):
                best, jlo, jhi = carry
                bl2, br2 = bounds(jlo, jhi)
                go_left = bl2 <= br2
                t = jnp.where(go_left, jlo - 1, jhi + 1)
                best = strip_pass(t, best, qx, qy)
                return (best, jnp.where(go_left, jlo - 1, jlo),
                        jnp.where(go_left, jhi, jhi + 1))

            (bd, bi), _, _ = lax.while_loop(cond, expand,
                                            ((bd, bi), s0, s0))

            gx = plsc.load_gather(x_v, [bi])
            gy = plsc.load_gather(y_v, [bi])
            gz = plsc.load_gather(z_v, [bi])
            slot = lane_ppt + p
            plsc.store_scatter(nx_v, [slot], gx)
            plsc.store_scatter(ny_v, [slot], gy)
            plsc.store_scatter(nz_v, [slot], gz)
            return 0

        lax.fori_loop(0, ppt, point_body, 0)
        for j in range(16):
            sl = pl.ds(j * ppt, ppt)
            pltpu.sync_copy(nx_v.at[sl], ox_hbm.at[pl.ds(j * N + base, ppt)])
            pltpu.sync_copy(ny_v.at[sl], oy_hbm.at[pl.ds(j * N + base, ppt)])
            pltpu.sync_copy(nz_v.at[sl], oz_hbm.at[pl.ds(j * N + base, ppt)])

    return body(ux, uy, x, y, z)


def _cov_eig_kernel(nx_ref, ny_ref, nz_ref, out_ref):
    k = jnp.float32(_K)
    gx = nx_ref[...]
    gy = ny_ref[...]
    gz = nz_ref[...]
    sx = jnp.sum(gx, axis=0, keepdims=True)
    sy = jnp.sum(gy, axis=0, keepdims=True)
    sz = jnp.sum(gz, axis=0, keepdims=True)
    c00 = jnp.sum(gx * gx, axis=0, keepdims=True) - sx * sx / k
    c11 = jnp.sum(gy * gy, axis=0, keepdims=True) - sy * sy / k
    c22 = jnp.sum(gz * gz, axis=0, keepdims=True) - sz * sz / k
    c01 = jnp.sum(gx * gy, axis=0, keepdims=True) - sx * sy / k
    c02 = jnp.sum(gx * gz, axis=0, keepdims=True) - sx * sz / k
    c12 = jnp.sum(gy * gz, axis=0, keepdims=True) - sy * sz / k

    one = jnp.float32(1.0)
    zero = jnp.float32(0.0)

    def rot(app, aqq, apq, arp, arq):
        denom = 2.0 * apq
        theta = (aqq - app) / jnp.where(denom == zero, one, denom)
        sgn = jnp.where(theta >= zero, one, -one)
        t = sgn / (jnp.abs(theta) + jnp.sqrt(theta * theta + one))
        t = jnp.where(apq == zero, zero, t)
        c = lax.rsqrt(t * t + one)
        s = t * c
        napp = app - t * apq
        naqq = aqq + t * apq
        narp = c * arp - s * arq
        narq = s * arp + c * arq
        return napp, naqq, narp, narq

    a00, a11, a22, a01, a02, a12 = c00, c11, c22, c01, c02, c12
    for _ in range(6):
        a00, a11, a02, a12 = rot(a00, a11, a01, a02, a12)
        a01 = zero * a01
        a00, a22, a01, a12 = rot(a00, a22, a02, a01, a12)
        a02 = zero * a02
        a11, a22, a01, a02 = rot(a11, a22, a12, a01, a02)
        a12 = zero * a12

    e1 = jnp.maximum(jnp.maximum(a00, a11), a22)
    e3 = jnp.minimum(jnp.minimum(a00, a11), a22)
    e2 = (a00 + a11 + a22) - e1 - e3
    out_ref[0:1, :] = e1
    out_ref[1:2, :] = e2
    out_ref[2:3, :] = e3


@jax.jit
def kernel(X, uv):
    B, M, _ = X.shape
    N = B * M
    nx, ny, nz = _sc_stage1(uv[..., 0].reshape(-1), uv[..., 1].reshape(-1),
                            X[..., 0].reshape(-1), X[..., 1].reshape(-1),
                            X[..., 2].reshape(-1), B, M)
    nx = nx.reshape(16, N)
    ny = ny.reshape(16, N)
    nz = nz.reshape(16, N)
    L = 2048
    eig = pl.pallas_call(
        _cov_eig_kernel,
        grid=(N // L,),
        in_specs=[pl.BlockSpec((16, L), lambda i: (0, i))] * 3,
        out_specs=pl.BlockSpec((3, L), lambda i: (0, i)),
        out_shape=jax.ShapeDtypeStruct((3, N), jnp.float32),
    )(nx, ny, nz)
    return eig.T.reshape(B, M, 3)


# speedup vs baseline: 1.1676x; 1.1676x over previous
import functools

import jax
import jax.numpy as jnp
from jax import lax
from jax.experimental import pallas as pl
from jax.experimental.pallas import tpu as pltpu
from jax.experimental.pallas import tpu_sc as plsc


_K = 16
_NC = 2
_NS = 16
_NW = _NC * _NS


def _sc_stage1(ux, uy, x, y, z, B, M):
    N = B * M
    ppt = N // _NW
    seg = M // ppt
    nchunk = M // 16

    @functools.partial(
        pl.kernel,
        mesh=plsc.VectorSubcoreMesh(core_axis_name="c", subcore_axis_name="s"),
        out_type=jax.ShapeDtypeStruct((N, 16), jnp.float32),
        compiler_params=pltpu.CompilerParams(needs_layout_passes=False),
        scratch_types=[
            pltpu.VMEM((M,), jnp.float32),
            pltpu.VMEM((M,), jnp.float32),
            pltpu.VMEM((M,), jnp.float32),
            pltpu.VMEM((M,), jnp.float32),
            pltpu.VMEM((M,), jnp.float32),
            pltpu.VMEM((ppt, 16), jnp.float32),
        ],
    )
    def body(ux_hbm, uy_hbm, x_hbm, y_hbm, z_hbm, out_hbm,
             ux_v, uy_v, x_v, y_v, z_v, mom_v):
        cid = lax.axis_index("c")
        sid = lax.axis_index("s")
        wid = sid * _NC + cid
        base = wid * ppt
        boff = (wid // seg) * M
        pltpu.sync_copy(ux_hbm.at[pl.ds(boff, M)], ux_v)
        pltpu.sync_copy(uy_hbm.at[pl.ds(boff, M)], uy_v)
        pltpu.sync_copy(x_hbm.at[pl.ds(boff, M)], x_v)
        pltpu.sync_copy(y_hbm.at[pl.ds(boff, M)], y_v)
        pltpu.sync_copy(z_hbm.at[pl.ds(boff, M)], z_v)

        lane = lax.iota(jnp.int32, 16)
        zero16 = jnp.zeros((16,), jnp.int32)
        inf16 = jnp.full((16,), jnp.float32(jnp.inf))

        def point_body(p, _):
            loc16 = jnp.full((16,), (wid % seg) * ppt + p, jnp.int32)
            qx = plsc.load_gather(ux_v, [loc16])
            qy = plsc.load_gather(uy_v, [loc16])

            def merge(a, b, descending):
                take = b[0] < a[0]
                nd = jnp.where(take, b[0], a[0])
                ni = jnp.where(take, b[1], a[1])
                nk, nv = plsc.sort_key_val(nd, ni, descending=descending)
                return (nk, nv)

            def chunk_body(ci, carry):
                off0 = pl.multiple_of(ci * 256, 256)
                level = []
                for j in range(16):
                    off = off0 + j * 16
                    cx = ux_v[pl.ds(off, 16)]
                    cy = uy_v[pl.ds(off, 16)]
                    du = cx - qx
                    dv = cy - qy
                    d = du * du + dv * dv
                    sd, si = plsc.sort_key_val(d, lane + off,
                                               descending=(j % 2 == 1))
                    level.append((sd, si))
                while len(level) > 2:
                    level = [merge(level[i], level[i + 1],
                                   descending=(i % 4 == 2))
                             for i in range(0, len(level), 2)]
                root = merge(level[0], level[1], descending=True)
                return merge(carry, root, descending=False)

            _, bi = lax.fori_loop(0, nchunk // 16, chunk_body,
                                  (inf16, zero16))
            gx = plsc.load_gather(x_v, [bi])
            gy = plsc.load_gather(y_v, [bi])
            gz = plsc.load_gather(z_v, [bi])
            sums = (gx, gy, gz, gx * gx, gy * gy, gz * gz,
                    gx * gy, gx * gz, gy * gz)
            mom = jnp.zeros((16,), jnp.float32)
            for j, v in enumerate(sums):
                mom = mom + jnp.where(lane == j, jnp.sum(v), jnp.float32(0.0))
            mom_v[p] = mom
            return 0

        lax.fori_loop(0, ppt, point_body, 0)
        pltpu.sync_copy(mom_v, out_hbm.at[pl.ds(base, ppt)])

    return body(ux, uy, x, y, z)


def _eig_kernel(m_ref, out_ref):
    k = jnp.float32(_K)
    sx = m_ref[0:1, :]
    sy = m_ref[1:2, :]
    sz = m_ref[2:3, :]
    c00 = m_ref[3:4, :] - sx * sx / k
    c11 = m_ref[4:5, :] - sy * sy / k
    c22 = m_ref[5:6, :] - sz * sz / k
    c01 = m_ref[6:7, :] - sx * sy / k
    c02 = m_ref[7:8, :] - sx * sz / k
    c12 = m_ref[8:9, :] - sy * sz / k

    one = jnp.float32(1.0)
    zero = jnp.float32(0.0)

    def rot(app, aqq, apq, arp, arq):
        denom = 2.0 * apq
        theta = (aqq - app) / jnp.where(denom == zero, one, denom)
        sgn = jnp.where(theta >= zero, one, -one)
        t = sgn / (jnp.abs(theta) + jnp.sqrt(theta * theta + one))
        t = jnp.where(apq == zero, zero, t)
        c = lax.rsqrt(t * t + one)
        s = t * c
        napp = app - t * apq
        naqq = aqq + t * apq
        narp = c * arp - s * arq
        narq = s * arp + c * arq
        return napp, naqq, narp, narq

    a00, a11, a22, a01, a02, a12 = c00, c11, c22, c01, c02, c12
    for _ in range(6):
        a00, a11, a02, a12 = rot(a00, a11, a01, a02, a12)
        a01 = zero * a01
        a00, a22, a01, a12 = rot(a00, a22, a02, a01, a12)
        a02 = zero * a02
        a11, a22, a01, a02 = rot(a11, a22, a12, a01, a02)
        a12 = zero * a12

    e1 = jnp.maximum(jnp.maximum(a00, a11), a22)
    e3 = jnp.minimum(jnp.minimum(a00, a11), a22)
    e2 = (a00 + a11 + a22) - e1 - e3
    out_ref[0:1, :] = e1
    out_ref[1:2, :] = e2
    out_ref[2:3, :] = e3


@jax.jit
def kernel(X, uv):
    B, M, _ = X.shape
    mom16 = _sc_stage1(uv[..., 0].reshape(-1), uv[..., 1].reshape(-1),
                       X[..., 0].reshape(-1), X[..., 1].reshape(-1),
                       X[..., 2].reshape(-1), B, M)
    m9 = mom16[:, :9].T
    eig = pl.pallas_call(
        _eig_kernel,
        out_shape=jax.ShapeDtypeStruct((3, B * M), jnp.float32),
    )(m9)
    return eig.T.reshape(B, M, 3)


# speedup vs baseline: 1.1711x; 1.0030x over previous
import functools

import jax
import jax.numpy as jnp
from jax import lax
from jax.experimental import pallas as pl
from jax.experimental.pallas import tpu as pltpu
from jax.experimental.pallas import tpu_sc as plsc


_K = 16
_NC = 2
_NS = 16
_NW = _NC * _NS


def _sc_stage1(ux, uy, x, y, z, B, M):
    N = B * M
    ppt = N // _NW
    seg = M // ppt
    nchunk = M // 16

    @functools.partial(
        pl.kernel,
        mesh=plsc.VectorSubcoreMesh(core_axis_name="c", subcore_axis_name="s"),
        out_type=jax.ShapeDtypeStruct((N, 16), jnp.float32),
        compiler_params=pltpu.CompilerParams(needs_layout_passes=False),
        scratch_types=[
            pltpu.VMEM((M,), jnp.float32),
            pltpu.VMEM((M,), jnp.float32),
            pltpu.VMEM((M,), jnp.float32),
            pltpu.VMEM((M,), jnp.float32),
            pltpu.VMEM((M,), jnp.float32),
            pltpu.VMEM((ppt, 16), jnp.float32),
        ],
    )
    def body(ux_hbm, uy_hbm, x_hbm, y_hbm, z_hbm, out_hbm,
             ux_v, uy_v, x_v, y_v, z_v, mom_v):
        cid = lax.axis_index("c")
        sid = lax.axis_index("s")
        wid = sid * _NC + cid
        base = wid * ppt
        boff = (wid // seg) * M
        pltpu.sync_copy(ux_hbm.at[pl.ds(boff, M)], ux_v)
        pltpu.sync_copy(uy_hbm.at[pl.ds(boff, M)], uy_v)
        pltpu.sync_copy(x_hbm.at[pl.ds(boff, M)], x_v)
        pltpu.sync_copy(y_hbm.at[pl.ds(boff, M)], y_v)
        pltpu.sync_copy(z_hbm.at[pl.ds(boff, M)], z_v)

        lane = lax.iota(jnp.int32, 16)
        zero16 = jnp.zeros((16,), jnp.int32)
        inf16 = jnp.full((16,), jnp.float32(jnp.inf))

        def point_body(p, _):
            loc16 = jnp.full((16,), (wid % seg) * ppt + p, jnp.int32)
            qx = plsc.load_gather(ux_v, [loc16])
            qy = plsc.load_gather(uy_v, [loc16])

            def merge(a, b, descending):
                take = b[0] < a[0]
                nd = jnp.where(take, b[0], a[0])
                ni = jnp.where(take, b[1], a[1])
                nk, nv = plsc.sort_key_val(nd, ni, descending=descending)
                return (nk, nv)

            def chunk_body(ci, carry):
                off0 = pl.multiple_of(ci * 256, 256)
                level = []
                for j in range(16):
                    off = off0 + j * 16
                    cx = ux_v[pl.ds(off, 16)]
                    cy = uy_v[pl.ds(off, 16)]
                    du = cx - qx
                    dv = cy - qy
                    d = du * du + dv * dv
                    sd, si = plsc.sort_key_val(d, lane + off,
                                               descending=(j % 2 == 1))
                    level.append((sd, si))
                while len(level) > 2:
                    level = [merge(level[i], level[i + 1],
                                   descending=(i % 4 == 2))
                             for i in range(0, len(level), 2)]
                root = merge(level[0], level[1], descending=True)
                return merge(carry, root, descending=False)

            _, bi = lax.fori_loop(0, nchunk // 16, chunk_body,
                                  (inf16, zero16))
            gx = plsc.load_gather(x_v, [bi])
            gy = plsc.load_gather(y_v, [bi])
            gz = plsc.load_gather(z_v, [bi])
            sums = (gx, gy, gz, gx * gx, gy * gy, gz * gz,
                    gx * gy, gx * gz, gy * gz)
            mom = jnp.zeros((16,), jnp.float32)
            for j, v in enumerate(sums):
                mom = mom + jnp.where(lane == j, jnp.sum(v), jnp.float32(0.0))
            mom_v[p] = mom
            return 0

        lax.fori_loop(0, ppt, point_body, 0)
        pltpu.sync_copy(mom_v, out_hbm.at[pl.ds(base, ppt)])

    return body(ux, uy, x, y, z)


def _eig_kernel(m_ref, out_ref):
    k = jnp.float32(_K)
    sx = m_ref[0:1, :]
    sy = m_ref[1:2, :]
    sz = m_ref[2:3, :]
    c00 = m_ref[3:4, :] - sx * sx / k
    c11 = m_ref[4:5, :] - sy * sy / k
    c22 = m_ref[5:6, :] - sz * sz / k
    c01 = m_ref[6:7, :] - sx * sy / k
    c02 = m_ref[7:8, :] - sx * sz / k
    c12 = m_ref[8:9, :] - sy * sz / k

    one = jnp.float32(1.0)
    zero = jnp.float32(0.0)

    def rot(app, aqq, apq, arp, arq):
        denom = 2.0 * apq
        theta = (aqq - app) / jnp.where(denom == zero, one, denom)
        sgn = jnp.where(theta >= zero, one, -one)
        t = sgn / (jnp.abs(theta) + jnp.sqrt(theta * theta + one))
        t = jnp.where(apq == zero, zero, t)
        c = lax.rsqrt(t * t + one)
        s = t * c
        napp = app - t * apq
        naqq = aqq + t * apq
        narp = c * arp - s * arq
        narq = s * arp + c * arq
        return napp, naqq, narp, narq

    a00, a11, a22, a01, a02, a12 = c00, c11, c22, c01, c02, c12
    for _ in range(4):
        a00, a11, a02, a12 = rot(a00, a11, a01, a02, a12)
        a01 = zero * a01
        a00, a22, a01, a12 = rot(a00, a22, a02, a01, a12)
        a02 = zero * a02
        a11, a22, a01, a02 = rot(a11, a22, a12, a01, a02)
        a12 = zero * a12

    e1 = jnp.maximum(jnp.maximum(a00, a11), a22)
    e3 = jnp.minimum(jnp.minimum(a00, a11), a22)
    e2 = (a00 + a11 + a22) - e1 - e3
    out_ref[0:1, :] = e1
    out_ref[1:2, :] = e2
    out_ref[2:3, :] = e3


@jax.jit
def kernel(X, uv):
    B, M, _ = X.shape
    mom16 = _sc_stage1(uv[..., 0].reshape(-1), uv[..., 1].reshape(-1),
                       X[..., 0].reshape(-1), X[..., 1].reshape(-1),
                       X[..., 2].reshape(-1), B, M)
    m9 = mom16[:, :9].T
    eig = pl.pallas_call(
        _eig_kernel,
        out_shape=jax.ShapeDtypeStruct((3, B * M), jnp.float32),
    )(m9)
    return eig.T.reshape(B, M, 3)
